# Initial kernel scaffold; baseline (speedup 1.0000x reference)
import functools

import jax
import jax.numpy as jnp
from jax import lax
from jax.experimental import pallas as pl
from jax.experimental.pallas import tpu as pltpu
from jax.experimental.pallas import tpu_sc as plsc

NUM_CORES = 2
NUM_SUBCORES = 16
NUM_WORKERS = NUM_CORES * NUM_SUBCORES
LANES = 16

B = 4096 * 200
D = 25
VOCAB = 100000
B_PER_W = B // NUM_WORKERS
CHUNK = 1600
N_CHUNKS = B_PER_W // CHUNK


@functools.partial(
    pl.kernel,
    out_type=jax.ShapeDtypeStruct((B, D), jnp.float32),
    mesh=plsc.VectorSubcoreMesh(
        core_axis_name="c", subcore_axis_name="s",
        num_cores=NUM_CORES, num_subcores=NUM_SUBCORES),
    scratch_types=[
        pltpu.VMEM((CHUNK,), jnp.int32),
        pltpu.VMEM((CHUNK, D), jnp.float32),
        pltpu.SemaphoreType.DMA,
    ],
)
def _gather_kernel(table_hbm, idx_hbm, out_hbm, idx_v, rows_v, sem):
    wid = lax.axis_index("s") * NUM_CORES + lax.axis_index("c")
    base = wid * B_PER_W

    def chunk_body(c, _):
        off = base + c * CHUNK
        pltpu.sync_copy(idx_hbm.at[pl.ds(off, CHUNK)], idx_v)

        def clamp_body(i, _):
            v = idx_v[pl.ds(i * LANES, LANES)]
            idx_v[pl.ds(i * LANES, LANES)] = jnp.where(v >= VOCAB, 0, v)
            return ()

        lax.fori_loop(0, CHUNK // LANES, clamp_body, (), unroll=4)

        pltpu.async_copy(table_hbm.at[idx_v], rows_v, sem).wait()
        pltpu.sync_copy(rows_v, out_hbm.at[pl.ds(off, CHUNK)])
        return ()

    lax.fori_loop(0, N_CHUNKS, chunk_body, ())


def kernel(x, table):
    idx = x.reshape(-1).astype(jnp.int32)
    out = _gather_kernel(table, idx)
    return out.reshape(x.shape + (D,))


# speedup vs baseline: 3.5918x; 3.5918x over previous
import functools

import jax
import jax.numpy as jnp
from jax import lax
from jax.experimental import pallas as pl
from jax.experimental.pallas import tpu as pltpu
from jax.experimental.pallas import tpu_sc as plsc

NUM_CORES = 2
NUM_SUBCORES = 16
NUM_WORKERS = NUM_CORES * NUM_SUBCORES
LANES = 16

B = 4096 * 200
D = 25
DP = 32
VOCAB = 100000
G = 128
GROUPS = B // G
G_PER_W = GROUPS // NUM_WORKERS


@functools.partial(
    pl.kernel,
    out_type=jax.ShapeDtypeStruct((B, DP), jnp.float32),
    mesh=plsc.VectorSubcoreMesh(
        core_axis_name="c", subcore_axis_name="s",
        num_cores=NUM_CORES, num_subcores=NUM_SUBCORES),
    scratch_types=[
        pltpu.VMEM((G,), jnp.int32),
        pltpu.VMEM((G, DP), jnp.float32),
        pltpu.SemaphoreType.DMA,
    ],
    compiler_params=pltpu.CompilerParams(use_tc_tiling_on_sc=False),
)
def _gather_kernel(table_hbm, idx_hbm, out_hbm, idx_v, rows_v, sem):
    wid = lax.axis_index("s") * NUM_CORES + lax.axis_index("c")
    gbase = wid * G_PER_W

    def group_body(g, _):
        off = (gbase + g) * G
        pltpu.sync_copy(idx_hbm.at[pl.ds(off, G)], idx_v)
        pltpu.async_copy(table_hbm.at[idx_v], rows_v, sem).wait()
        pltpu.sync_copy(rows_v, out_hbm.at[pl.ds(off, G)])
        return ()

    lax.fori_loop(0, G_PER_W, group_body, ())


def kernel(x, table):
    idx = x.reshape(-1).astype(jnp.int32)
    table_p = jnp.pad(table, ((0, 0), (0, DP - D)))
    out = _gather_kernel(table_p, idx)
    return out[:, :D].reshape(x.shape + (D,))


# speedup vs baseline: 4.9237x; 1.3708x over previous
import functools

import jax
import jax.numpy as jnp
from jax import lax
from jax.experimental import pallas as pl
from jax.experimental.pallas import tpu as pltpu
from jax.experimental.pallas import tpu_sc as plsc

NUM_CORES = 2
NUM_SUBCORES = 16
NUM_WORKERS = NUM_CORES * NUM_SUBCORES

B = 4096 * 200
D = 25
DP = 32
VOCAB = 100000
G = 128
K = 10
BATCH = K * G
GROUPS = B // G
G_PER_W = GROUPS // NUM_WORKERS
NB = G_PER_W // K
NI = NB // 2


@functools.partial(
    pl.kernel,
    out_type=jax.ShapeDtypeStruct((B, DP), jnp.float32),
    mesh=plsc.VectorSubcoreMesh(
        core_axis_name="c", subcore_axis_name="s",
        num_cores=NUM_CORES, num_subcores=NUM_SUBCORES),
    scratch_types=[
        pltpu.VMEM((BATCH,), jnp.int32),
        pltpu.VMEM((BATCH,), jnp.int32),
        pltpu.VMEM((BATCH, DP), jnp.float32),
        pltpu.VMEM((BATCH, DP), jnp.float32),
        pltpu.SemaphoreType.DMA,
        pltpu.SemaphoreType.DMA,
        pltpu.SemaphoreType.DMA,
        pltpu.SemaphoreType.DMA,
        pltpu.SemaphoreType.DMA,
        pltpu.SemaphoreType.DMA,
    ],
    compiler_params=pltpu.CompilerParams(use_tc_tiling_on_sc=False),
)
def _gather_kernel(table_hbm, idx_hbm, out_hbm,
                   idx0, idx1, rows0, rows1,
                   sem_i0, sem_i1, sem_g0, sem_g1, sem_o0, sem_o1):
    wid = lax.axis_index("s") * NUM_CORES + lax.axis_index("c")
    row_base = wid * G_PER_W * G

    def fire_idx(b, idx_v, sem):
        return pltpu.async_copy(
            idx_hbm.at[pl.ds(row_base + b * BATCH, BATCH)], idx_v, sem)

    def wait_idx(idx_v, sem):
        pltpu.make_async_copy(idx_hbm.at[pl.ds(0, BATCH)], idx_v, sem).wait()

    def fire_gathers(idx_v, rows_v, sem):
        return [
            pltpu.async_copy(
                table_hbm.at[idx_v.at[pl.ds(j * G, G)]],
                rows_v.at[pl.ds(j * G, G)],
                sem,
            )
            for j in range(K)
        ]

    def fire_out(b, rows_v, sem):
        return pltpu.async_copy(
            rows_v, out_hbm.at[pl.ds(row_base + b * BATCH, BATCH)], sem)

    def drain_out(rows_v, sem):
        pltpu.make_async_copy(rows_v, out_hbm.at[pl.ds(0, BATCH)], sem).wait()

    fire_idx(0, idx0, sem_i0)

    def body(i, _):
        b0 = 2 * i
        b1 = b0 + 1

        @pl.when(i > 0)
        def _():
            drain_out(rows0, sem_o0)
            drain_out(rows1, sem_o1)

        wait_idx(idx0, sem_i0)
        g0 = fire_gathers(idx0, rows0, sem_g0)
        fire_idx(b1, idx1, sem_i1)
        for cp in g0:
            cp.wait()
        fire_out(b0, rows0, sem_o0)

        @pl.when(i < NI - 1)
        def _():
            fire_idx(b0 + 2, idx0, sem_i0)

        wait_idx(idx1, sem_i1)
        g1 = fire_gathers(idx1, rows1, sem_g1)
        for cp in g1:
            cp.wait()
        fire_out(b1, rows1, sem_o1)
        return ()

    lax.fori_loop(0, NI, body, ())

    drain_out(rows0, sem_o0)
    drain_out(rows1, sem_o1)


def kernel(x, table):
    idx = x.reshape(-1).astype(jnp.int32)
    table_p = jnp.pad(table, ((0, 0), (0, DP - D)))
    out = _gather_kernel(table_p, idx)
    return out[:, :D].reshape(x.shape + (D,))


# speedup vs baseline: 8.9372x; 1.8151x over previous
import functools

import jax
import jax.numpy as jnp
from jax import lax
from jax.experimental import pallas as pl
from jax.experimental.pallas import tpu as pltpu
from jax.experimental.pallas import tpu_sc as plsc

NUM_CORES = 2
NUM_SUBCORES = 16
NUM_WORKERS = NUM_CORES * NUM_SUBCORES

B = 4096 * 200
D = 25
DP = 32
VOCAB = 100000
G = 128
K = 10
BATCH = K * G
GROUPS = B // G
G_PER_W = GROUPS // NUM_WORKERS
NB = G_PER_W // K
NI = NB // 2


@functools.partial(
    pl.kernel,
    out_type=jax.ShapeDtypeStruct((B, 128), jnp.float32),
    mesh=plsc.VectorSubcoreMesh(
        core_axis_name="c", subcore_axis_name="s",
        num_cores=NUM_CORES, num_subcores=NUM_SUBCORES),
    scratch_types=[
        pltpu.VMEM((BATCH,), jnp.int32),
        pltpu.VMEM((BATCH,), jnp.int32),
        pltpu.VMEM((BATCH, DP), jnp.float32),
        pltpu.VMEM((BATCH, DP), jnp.float32),
        pltpu.SemaphoreType.DMA,
        pltpu.SemaphoreType.DMA,
        pltpu.SemaphoreType.DMA,
        pltpu.SemaphoreType.DMA,
        pltpu.SemaphoreType.DMA,
        pltpu.SemaphoreType.DMA,
    ],
    compiler_params=pltpu.CompilerParams(use_tc_tiling_on_sc=False),
)
def _gather_kernel(table_hbm, idx_hbm, out_hbm,
                   idx0, idx1, rows0, rows1,
                   sem_i0, sem_i1, sem_g0, sem_g1, sem_o0, sem_o1):
    wid = lax.axis_index("s") * NUM_CORES + lax.axis_index("c")
    row_base = wid * G_PER_W * G

    def fire_idx(b, idx_v, sem):
        return pltpu.async_copy(
            idx_hbm.at[pl.ds(row_base + b * BATCH, BATCH)], idx_v, sem)

    def wait_idx(idx_v, sem):
        pltpu.make_async_copy(idx_hbm.at[pl.ds(0, BATCH)], idx_v, sem).wait()

    def fire_gathers(idx_v, rows_v, sem):
        return [
            pltpu.async_copy(
                table_hbm.at[idx_v.at[pl.ds(j * G, G)]],
                rows_v.at[pl.ds(j * G, G)],
                sem,
            )
            for j in range(K)
        ]

    def fire_out(b, rows_v, sem):
        return pltpu.async_copy(
            rows_v,
            out_hbm.at[pl.ds(row_base + b * BATCH, BATCH), pl.ds(0, DP)],
            sem)

    def drain_out(rows_v, sem):
        pltpu.make_async_copy(
            rows_v, out_hbm.at[pl.ds(0, BATCH), pl.ds(0, DP)], sem).wait()

    fire_idx(0, idx0, sem_i0)

    def body(i, _):
        b0 = 2 * i
        b1 = b0 + 1

        @pl.when(i > 0)
        def _():
            drain_out(rows0, sem_o0)
            drain_out(rows1, sem_o1)

        wait_idx(idx0, sem_i0)
        g0 = fire_gathers(idx0, rows0, sem_g0)
        fire_idx(b1, idx1, sem_i1)
        for cp in g0:
            cp.wait()
        fire_out(b0, rows0, sem_o0)

        @pl.when(i < NI - 1)
        def _():
            fire_idx(b0 + 2, idx0, sem_i0)

        wait_idx(idx1, sem_i1)
        g1 = fire_gathers(idx1, rows1, sem_g1)
        for cp in g1:
            cp.wait()
        fire_out(b1, rows1, sem_o1)
        return ()

    lax.fori_loop(0, NI, body, ())

    drain_out(rows0, sem_o0)
    drain_out(rows1, sem_o1)


def kernel(x, table):
    idx = x.reshape(-1).astype(jnp.int32)
    table_p = jnp.pad(table, ((0, 0), (0, DP - D)))
    out = _gather_kernel(table_p, idx)
    return out[:, :D].reshape(x.shape + (D,))


# speedup vs baseline: 10.4218x; 1.1661x over previous
import functools

import jax
import jax.numpy as jnp
from jax import lax
from jax.experimental import pallas as pl
from jax.experimental.pallas import tpu as pltpu
from jax.experimental.pallas import tpu_sc as plsc

NUM_CORES = 2
NUM_SUBCORES = 16
NUM_WORKERS = NUM_CORES * NUM_SUBCORES

B = 4096 * 200
D = 25
DP = 32
VOCAB = 100000
G = 128
K = 10
BATCH = K * G
GROUPS = B // G
G_PER_W = GROUPS // NUM_WORKERS
NB = G_PER_W // K
NI = NB // 2


@functools.partial(
    pl.kernel,
    out_type=jax.ShapeDtypeStruct((B, 128), jnp.float32),
    mesh=plsc.VectorSubcoreMesh(
        core_axis_name="c", subcore_axis_name="s",
        num_cores=NUM_CORES, num_subcores=NUM_SUBCORES),
    scratch_types=[
        pltpu.VMEM((BATCH,), jnp.int32),
        pltpu.VMEM((BATCH,), jnp.int32),
        pltpu.VMEM((BATCH, DP), jnp.float32),
        pltpu.VMEM((BATCH, DP), jnp.float32),
        pltpu.SemaphoreType.DMA,
        pltpu.SemaphoreType.DMA,
        pltpu.SemaphoreType.DMA,
        pltpu.SemaphoreType.DMA,
        pltpu.SemaphoreType.DMA,
        pltpu.SemaphoreType.DMA,
    ],
    compiler_params=pltpu.CompilerParams(use_tc_tiling_on_sc=False),
)
def _gather_kernel(table_hbm, idx_hbm, out_hbm,
                   idx0, idx1, rows0, rows1,
                   sem_i0, sem_i1, sem_g0, sem_g1, sem_o0, sem_o1):
    wid = lax.axis_index("s") * NUM_CORES + lax.axis_index("c")
    row_base = wid * G_PER_W * G

    def fire_idx(b, idx_v, sem):
        return pltpu.async_copy(
            idx_hbm.at[pl.ds(row_base + b * BATCH, BATCH)], idx_v, sem)

    def wait_idx(idx_v, sem):
        pltpu.make_async_copy(idx_hbm.at[pl.ds(0, BATCH)], idx_v, sem).wait()

    def fire_gathers(idx_v, rows_v, sem):
        return [
            pltpu.async_copy(
                table_hbm.at[idx_v.at[pl.ds(j * G, G)]],
                rows_v.at[pl.ds(j * G, G)],
                sem,
            )
            for j in range(K)
        ]

    def fire_out(b, rows_v, sem):
        return pltpu.async_copy(
            rows_v,
            out_hbm.at[pl.ds(row_base + b * BATCH, BATCH), pl.ds(0, DP)],
            sem)

    def drain_out(rows_v, sem):
        pltpu.make_async_copy(
            rows_v, out_hbm.at[pl.ds(0, BATCH), pl.ds(0, DP)], sem).wait()

    fire_idx(0, idx0, sem_i0)

    def body(i, _):
        b0 = 2 * i
        b1 = b0 + 1

        @pl.when(i > 0)
        def _():
            drain_out(rows0, sem_o0)
            drain_out(rows1, sem_o1)

        wait_idx(idx0, sem_i0)
        g0 = fire_gathers(idx0, rows0, sem_g0)
        fire_idx(b1, idx1, sem_i1)
        for cp in g0:
            cp.wait()
        fire_out(b0, rows0, sem_o0)

        @pl.when(i < NI - 1)
        def _():
            fire_idx(b0 + 2, idx0, sem_i0)

        wait_idx(idx1, sem_i1)
        g1 = fire_gathers(idx1, rows1, sem_g1)
        for cp in g1:
            cp.wait()
        fire_out(b1, rows1, sem_o1)
        return ()

    lax.fori_loop(0, NI, body, ())

    drain_out(rows0, sem_o0)
    drain_out(rows1, sem_o1)


def kernel(x, table):
    idx = x.T.reshape(-1).astype(jnp.int32)
    table_p = jnp.pad(table, ((0, 0), (0, DP - D)))
    out = _gather_kernel(table_p, idx)
    n_r, n_c = x.shape
    return out[:, :D].reshape(n_c, n_r, D).transpose(1, 0, 2)


# speedup vs baseline: 10.5579x; 1.0131x over previous
import functools

import jax
import jax.numpy as jnp
from jax import lax
from jax.experimental import pallas as pl
from jax.experimental.pallas import tpu as pltpu
from jax.experimental.pallas import tpu_sc as plsc

NUM_CORES = 2
NUM_SUBCORES = 16
NUM_WORKERS = NUM_CORES * NUM_SUBCORES

B = 4096 * 200
D = 25
DP = 32
VOCAB = 100000
G = 128
K = 10
BATCH = K * G
GROUPS = B // G
G_PER_W = GROUPS // NUM_WORKERS
NB = G_PER_W // K
NI = NB // 2


@functools.partial(
    pl.kernel,
    out_type=jax.ShapeDtypeStruct((B, 128), jnp.float32),
    mesh=plsc.VectorSubcoreMesh(
        core_axis_name="c", subcore_axis_name="s",
        num_cores=NUM_CORES, num_subcores=NUM_SUBCORES),
    scratch_types=[
        pltpu.VMEM((BATCH,), jnp.int32),
        pltpu.VMEM((BATCH,), jnp.int32),
        pltpu.VMEM((BATCH, DP), jnp.float32),
        pltpu.VMEM((BATCH, DP), jnp.float32),
        pltpu.SemaphoreType.DMA,
        pltpu.SemaphoreType.DMA,
        pltpu.SemaphoreType.DMA,
        pltpu.SemaphoreType.DMA,
        pltpu.SemaphoreType.DMA,
        pltpu.SemaphoreType.DMA,
    ],
    compiler_params=pltpu.CompilerParams(use_tc_tiling_on_sc=False),
)
def _gather_kernel(table_hbm, idx_hbm, out_hbm,
                   idx0, idx1, rows0, rows1,
                   sem_i0, sem_i1, sem_g0, sem_g1, sem_o0, sem_o1):
    wid = lax.axis_index("s") * NUM_CORES + lax.axis_index("c")
    row_base = wid * G_PER_W * G

    def fire_idx(b, idx_v, sem):
        return pltpu.async_copy(
            idx_hbm.at[pl.ds(row_base + b * BATCH, BATCH)], idx_v, sem)

    def wait_idx(idx_v, sem):
        pltpu.make_async_copy(idx_hbm.at[pl.ds(0, BATCH)], idx_v, sem).wait()

    def fire_gathers(idx_v, rows_v, sem):
        return [
            pltpu.async_copy(
                table_hbm.at[idx_v.at[pl.ds(j * G, G)]],
                rows_v.at[pl.ds(j * G, G)],
                sem,
            )
            for j in range(K)
        ]

    def fire_out(b, rows_v, sem):
        return pltpu.async_copy(
            rows_v,
            out_hbm.at[pl.ds(row_base + b * BATCH, BATCH), pl.ds(0, DP)],
            sem)

    def drain_out(rows_v, sem):
        pltpu.make_async_copy(
            rows_v, out_hbm.at[pl.ds(0, BATCH), pl.ds(0, DP)], sem).wait()

    fire_idx(0, idx0, sem_i0)
    fire_idx(1, idx1, sem_i1)

    def body(i, _):
        b0 = 2 * i
        b1 = b0 + 1

        @pl.when(i > 0)
        def _():
            drain_out(rows0, sem_o0)
        wait_idx(idx0, sem_i0)
        g0 = fire_gathers(idx0, rows0, sem_g0)

        @pl.when(i > 0)
        def _():
            drain_out(rows1, sem_o1)
        wait_idx(idx1, sem_i1)
        g1 = fire_gathers(idx1, rows1, sem_g1)

        for cp in g0:
            cp.wait()
        fire_out(b0, rows0, sem_o0)

        @pl.when(i < NI - 1)
        def _():
            fire_idx(b0 + 2, idx0, sem_i0)

        for cp in g1:
            cp.wait()
        fire_out(b1, rows1, sem_o1)

        @pl.when(i < NI - 1)
        def _():
            fire_idx(b1 + 2, idx1, sem_i1)
        return ()

    lax.fori_loop(0, NI, body, ())

    drain_out(rows0, sem_o0)
    drain_out(rows1, sem_o1)


def kernel(x, table):
    idx = x.T.reshape(-1).astype(jnp.int32)
    table_p = jnp.pad(table, ((0, 0), (0, DP - D)))
    out = _gather_kernel(table_p, idx)
    n_r, n_c = x.shape
    return out[:, :D].reshape(n_c, n_r, D).transpose(1, 0, 2)
